# baseline (speedup 1.0000x reference)
import functools

import jax
import jax.numpy as jnp
from jax import lax
from jax.experimental import pallas as pl
from jax.experimental.pallas import tpu as pltpu
from jax.experimental.pallas import tpu_sc as plsc

_B = 1024
_L = 200
_D = 64
_MAX_WAVELENGTH = 10000

_NC, _NS = 2, 16
_NW = _NC * _NS

_FLAT = _B * _L
_R = _FLAT // _NW
_CH = 2 * _L
_NCH = _R // _CH
_GSUB = 80
_NG = _CH // _GSUB
_NBUF = 3
_NQ = _D // 16


def _pos_encoding(seq_len, d, dtype=jnp.float32):
    i = jnp.arange(d)
    timescales = jnp.power(
        jnp.asarray(1.0 / _MAX_WAVELENGTH, dtype=dtype),
        (2 * (i // 2)).astype(dtype) / jnp.asarray(d, dtype=dtype))
    positions_mask = (i % 2).astype(dtype)
    positions = jnp.arange(seq_len).astype(dtype)
    angles = positions[:, None] * timescales[None, :]
    return jnp.sin(angles) * (1.0 - positions_mask) + jnp.cos(angles) * positions_mask


def _sc_body(idx_hbm, table_hbm, enc_hbm, out_hbm, idx_v, buf_v, enc_v, gsem, osem):
    w = lax.axis_index("s") * _NC + lax.axis_index("c")
    base = w * _R
    pltpu.sync_copy(idx_hbm.at[pl.ds(base, _R)], idx_v)
    pltpu.sync_copy(enc_hbm, enc_v)

    def fire_gathers(c, slot):
        descs = []
        for g in range(_NG):
            off = c * _CH + g * _GSUB
            descs.append(pltpu.async_copy(
                table_hbm.at[idx_v.at[pl.ds(off, _GSUB)]],
                buf_v.at[slot, pl.ds(g * _GSUB, _GSUB)],
                gsem))
        return descs

    gdescs = {0: fire_gathers(0, 0)}
    odescs = {}
    for c in range(_NCH):
        slot = c % _NBUF
        if c + 1 < _NCH:
            nslot = (c + 1) % _NBUF
            if c + 1 - _NBUF in odescs:
                odescs.pop(c + 1 - _NBUF).wait()
            gdescs[c + 1] = fire_gathers(c + 1, nslot)
        for d in gdescs.pop(c):
            d.wait()

        for half in range(_CH // _L):
            @plsc.parallel_loop(0, _L, unroll=4)
            def _(r, _half=half, _slot=slot):
                row = _half * _L + r
                for q in range(_NQ):
                    sl = pl.ds(q * 16, 16)
                    buf_v[_slot, row, sl] = buf_v[_slot, row, sl] + enc_v[r, sl]

        odescs[c] = pltpu.async_copy(
            buf_v.at[slot], out_hbm.at[pl.ds(base + c * _CH, _CH)], osem)
    for c in sorted(odescs):
        odescs.pop(c).wait()


@jax.jit
def _tok_pos_encode(idx_flat, table, enc):
    mesh = plsc.VectorSubcoreMesh(core_axis_name="c", subcore_axis_name="s")
    run = pl.kernel(
        _sc_body,
        out_type=jax.ShapeDtypeStruct((_FLAT, _D), jnp.float32),
        mesh=mesh,
        scratch_types=[
            pltpu.VMEM((_R,), jnp.int32),
            pltpu.VMEM((_NBUF, _CH, _D), jnp.float32),
            pltpu.VMEM((_L, _D), jnp.float32),
            pltpu.SemaphoreType.DMA,
            pltpu.SemaphoreType.DMA,
        ],
        compiler_params=pltpu.CompilerParams(use_tc_tiling_on_sc=False),
    )
    return run(idx_flat, table, enc)


def kernel(inputs, table):
    idx_flat = inputs.reshape(_FLAT).astype(jnp.int32)
    enc = _pos_encoding(_L, _D, table.dtype)
    out = _tok_pos_encode(idx_flat, table, enc)
    return out.reshape(_B, _L, _D)


# speedup vs baseline: 1.0018x; 1.0018x over previous
import functools

import jax
import jax.numpy as jnp
from jax import lax
from jax.experimental import pallas as pl
from jax.experimental.pallas import tpu as pltpu
from jax.experimental.pallas import tpu_sc as plsc

_VOCAB = 1000000
_B = 1024
_L = 200
_D = 64
_MAX_WAVELENGTH = 10000

_NC, _NS = 2, 16
_NW = _NC * _NS

_FLAT = _B * _L
_R = _FLAT // _NW
_CH = 2 * _L
_NCH = _R // _CH
_GSUB = 80
_NG = _CH // _GSUB
_NBUF = 3
_NQ = _D // 16


def _pos_encoding(seq_len, d, dtype=jnp.float32):
    i = jnp.arange(d)
    timescales = jnp.power(
        jnp.asarray(1.0 / _MAX_WAVELENGTH, dtype=dtype),
        (2 * (i // 2)).astype(dtype) / jnp.asarray(d, dtype=dtype))
    positions_mask = (i % 2).astype(dtype)
    positions = jnp.arange(seq_len).astype(dtype)
    angles = positions[:, None] * timescales[None, :]
    return jnp.sin(angles) * (1.0 - positions_mask) + jnp.cos(angles) * positions_mask


def _sc_body(idx_hbm, table_hbm, enc_hbm, out_hbm, idx_v, buf_v, enc_v, gsem, osem):
    w = lax.axis_index("s") * _NC + lax.axis_index("c")
    base = w * _R
    pltpu.sync_copy(idx_hbm.at[pl.ds(base, _R)], idx_v)
    pltpu.sync_copy(enc_hbm, enc_v)

    def fire_gathers(c, slot):
        descs = []
        for g in range(_NG):
            off = c * _CH + g * _GSUB
            descs.append(pltpu.async_copy(
                table_hbm.at[idx_v.at[pl.ds(off, _GSUB)]],
                buf_v.at[slot, pl.ds(g * _GSUB, _GSUB)],
                gsem))
        return descs

    gdescs = {0: fire_gathers(0, 0)}
    odescs = {}
    for c in range(_NCH):
        slot = c % _NBUF
        if c + 1 < _NCH:
            nslot = (c + 1) % _NBUF
            if c + 1 - _NBUF in odescs:
                odescs.pop(c + 1 - _NBUF).wait()
            gdescs[c + 1] = fire_gathers(c + 1, nslot)
        for d in gdescs.pop(c):
            d.wait()

        for half in range(_CH // _L):
            @plsc.parallel_loop(0, _L, unroll=4)
            def _(r, _half=half, _slot=slot):
                row = _half * _L + r
                for q in range(_NQ):
                    sl = pl.ds(q * 16, 16)
                    buf_v[_slot, row, sl] = buf_v[_slot, row, sl] + enc_v[r, sl]

        odescs[c] = pltpu.async_copy(
            buf_v.at[slot], out_hbm.at[pl.ds(base + c * _CH, _CH)], osem)
    for c in sorted(odescs):
        odescs.pop(c).wait()


@jax.jit
def _tok_pos_encode(idx_flat, table, enc):
    mesh = plsc.VectorSubcoreMesh(core_axis_name="c", subcore_axis_name="s")
    run = pl.kernel(
        _sc_body,
        out_type=jax.ShapeDtypeStruct((_FLAT, _D), jnp.float32),
        mesh=mesh,
        scratch_types=[
            pltpu.VMEM((_R,), jnp.int32),
            pltpu.VMEM((_NBUF, _CH, _D), jnp.float32),
            pltpu.VMEM((_L, _D), jnp.float32),
            pltpu.SemaphoreType.DMA,
            pltpu.SemaphoreType.DMA,
        ],
        compiler_params=pltpu.CompilerParams(use_tc_tiling_on_sc=False),
    )
    return run(idx_flat, table, enc)


def kernel(inputs, table):
    idx_flat = inputs.reshape(_FLAT).astype(jnp.int32)
    enc = _pos_encoding(_L, _D, table.dtype)
    tab_lin = jax.lax.optimization_barrier(table.reshape(_VOCAB // 2, 2 * _D))
    out = _tok_pos_encode(idx_flat, tab_lin.reshape(_VOCAB, _D), enc)
    return out.reshape(_B, _L, _D)
